# baseline (speedup 1.0000x reference)
import jax
import jax.numpy as jnp
from jax.experimental import pallas as pl

N = 2048
F_IN = 16
F_OUT = 32
B = 32


def _body(g0, g1, xtb, x0, W0, W6, b2, R0w, R0b, R1w, R1b, out):
    c0 = g0[:, 0:1]
    c1 = g1[:, 0:1]
    cv = jnp.concatenate([c0, c1], axis=1)
    d0 = jnp.dot(g0[...], cv, preferred_element_type=jnp.float32)
    d1 = jnp.dot(g1[...], cv, preferred_element_type=jnp.float32)
    C = jnp.concatenate([cv, d0, d1], axis=1)
    a = jax.lax.dot_general(C, xtb[...], (((0,), (0,)), ((), ())),
                            preferred_element_type=jnp.float32)
    w6 = W6[...]
    w0 = W0[...]
    x0v = x0[...]
    rows = []
    for bi in range(B):
        yb = jnp.dot(x0v[bi:bi + 1, :], w0, preferred_element_type=jnp.float32)
        sb = a[:, bi * F_IN:(bi + 1) * F_IN]
        for t in range(6):
            yb = yb + jnp.dot(sb[t:t + 1, :], w6[t],
                              preferred_element_type=jnp.float32)
        rows.append(yb)
    y = jnp.concatenate(rows, axis=0)
    y = jax.nn.sigmoid(y + b2[...])
    h = jax.nn.sigmoid(jnp.dot(y, R0w[...],
                               preferred_element_type=jnp.float32) + R0b[...])
    z = jnp.dot(h, R1w[...], preferred_element_type=jnp.float32) + R1b[...]
    out[...] = 1.0 + 4.0 * jax.nn.sigmoid(z)


def kernel(x, G0, G1, W, b, R0_w, R0_b, R1_w, R1_b):
    xt = jnp.transpose(x, (1, 0, 2)).reshape(N, B * F_IN)
    out = pl.pallas_call(
        _body,
        out_shape=jax.ShapeDtypeStruct((B, 1), jnp.float32),
    )(G0, G1, xt, x[:, 0, :], W[0], W[1:7], b.reshape(1, F_OUT),
      R0_w, R0_b.reshape(1, 16), R1_w, R1_b.reshape(1, 1))
    return out


# speedup vs baseline: 1.0643x; 1.0643x over previous
import jax
import jax.numpy as jnp
from jax.experimental import pallas as pl
from jax.experimental.pallas import tpu as pltpu

N = 2048
F_IN = 16
F_OUT = 32
B = 32
NBLK = 2
NB = N // NBLK


def _body(g0, g1, xtb, cv, x0T, W0T, W6T, bcol, R0T, R0b, R1T, R1b, out, acc):
    i = pl.program_id(0)
    cvf = cv[...]
    d0 = jnp.dot(g0[...], cvf, preferred_element_type=jnp.float32)
    d1 = jnp.dot(g1[...], cvf, preferred_element_type=jnp.float32)
    cb = cv[pl.ds(i * NB, NB), :]
    C = jnp.concatenate([cb, d0, d1], axis=1)
    part = jax.lax.dot_general(C, xtb[...], (((0,), (0,)), ((), ())),
                               preferred_element_type=jnp.float32)

    @pl.when(i == 0)
    def _init():
        acc[...] = part

    @pl.when(i > 0)
    def _accum():
        acc[...] += part

    @pl.when(i == NBLK - 1)
    def _final():
        a = acc[...]
        yT = jnp.dot(W0T[...], x0T[...],
                     preferred_element_type=jnp.float32)
        for f in range(F_IN):
            yT += jnp.dot(W6T[f], a[:, f * B:(f + 1) * B],
                          preferred_element_type=jnp.float32)
        yT = jax.nn.sigmoid(yT + bcol[...])
        h = jax.nn.sigmoid(jnp.dot(R0T[...], yT,
                                   preferred_element_type=jnp.float32) + R0b[...])
        z = jnp.dot(R1T[...], h,
                    preferred_element_type=jnp.float32) + R1b[...]
        out[...] = 1.0 + 4.0 * jax.nn.sigmoid(z)


def kernel(x, G0, G1, W, b, R0_w, R0_b, R1_w, R1_b):
    cvec = jnp.stack([G0[:, 0], G1[:, 0]], axis=1)
    xt = jnp.transpose(x, (1, 2, 0)).reshape(N, F_IN * B)
    x0T = x[:, 0, :].T
    W0T = W[0].T
    W6T = jnp.transpose(W[1:7], (1, 2, 0))
    bcol = b.reshape(F_OUT, 1)
    R0T = R0_w.T
    R0b = R0_b.reshape(16, 1)
    R1T = R1_w.T
    R1b = R1_b.reshape(1, 1)

    full = lambda s: pl.BlockSpec(s, lambda i: tuple(0 for _ in s))
    outT = pl.pallas_call(
        _body,
        grid=(NBLK,),
        in_specs=[
            pl.BlockSpec((NB, N), lambda i: (i, 0)),
            pl.BlockSpec((NB, N), lambda i: (i, 0)),
            pl.BlockSpec((NB, F_IN * B), lambda i: (i, 0)),
            full((N, 2)),
            full((F_IN, B)),
            full((F_OUT, F_IN)),
            full((F_IN, F_OUT, 6)),
            full((F_OUT, 1)),
            full((16, F_OUT)),
            full((16, 1)),
            full((1, 16)),
            full((1, 1)),
        ],
        out_specs=pl.BlockSpec((1, B), lambda i: (0, 0)),
        out_shape=jax.ShapeDtypeStruct((1, B), jnp.float32),
        scratch_shapes=[pltpu.VMEM((6, F_IN * B), jnp.float32)],
    )(G0, G1, xt, cvec, x0T, W0T, W6T, bcol, R0T, R0b, R1T, R1b)
    return outT.reshape(B, 1)


# speedup vs baseline: 1.0876x; 1.0218x over previous
import jax
import jax.numpy as jnp
from jax.experimental import pallas as pl

N = 2048
F_IN = 16
F_OUT = 32
B = 32


def _body(g0, g1, xtb, x0T, W0T, W6T, bcol, R0T, R0b, R1T, R1b, out):
    c0 = g0[:, 0:1]
    c1 = g1[:, 0:1]
    cv = jnp.concatenate([c0, c1], axis=1)
    d0 = jnp.dot(g0[...], cv, preferred_element_type=jnp.float32)
    d1 = jnp.dot(g1[...], cv, preferred_element_type=jnp.float32)
    zero2 = jnp.zeros((N, 2), jnp.float32)
    C = jnp.concatenate([cv, d0, d1, zero2], axis=1)
    a = jax.lax.dot_general(C, xtb[...], (((0,), (0,)), ((), ())),
                            preferred_element_type=jnp.float32)
    yT = jnp.dot(W0T[...], x0T[...],
                 preferred_element_type=jnp.float32)
    for f in range(F_IN):
        yT += jnp.dot(W6T[f], a[:, f * B:(f + 1) * B],  preferred_element_type=jnp.float32)
    yT = jax.nn.sigmoid(yT + bcol[...])
    h = jax.nn.sigmoid(jnp.dot(R0T[...], yT,           preferred_element_type=jnp.float32) + R0b[...])
    z = jnp.dot(R1T[...], h,
                preferred_element_type=jnp.float32) + R1b[...]
    out[...] = 1.0 + 4.0 * jax.nn.sigmoid(z)


def kernel(x, G0, G1, W, b, R0_w, R0_b, R1_w, R1_b):
    xt = jnp.transpose(x, (1, 2, 0)).reshape(N, F_IN * B)
    x0T = x[:, 0, :].T
    W0T = W[0].T
    W6T = jnp.concatenate(
        [jnp.transpose(W[1:7], (1, 2, 0)),
         jnp.zeros((F_IN, F_OUT, 2), jnp.float32)], axis=2)
    bcol = b.reshape(F_OUT, 1)
    R0T = R0_w.T
    R0b = R0_b.reshape(16, 1)
    R1T = R1_w.T
    R1b = R1_b.reshape(1, 1)

    outT = pl.pallas_call(
        _body,
        out_shape=jax.ShapeDtypeStruct((1, B), jnp.float32),
    )(G0, G1, xt, x0T, W0T, W6T, bcol, R0T, R0b, R1T, R1b)
    return outT.reshape(B, 1)
